# Initial kernel scaffold; baseline (speedup 1.0000x reference)
import jax
import jax.numpy as jnp
from jax.experimental import pallas as pl


def kernel(x, shared_prototypes, W_shared_to_class, rare_prototypes, rarity_factor):
    raise NotImplementedError("write your pallas kernel here")


# speedup vs baseline: 1.9540x; 1.9540x over previous
import math

import jax
import jax.numpy as jnp
from jax.experimental import pallas as pl

_B = 4096
_D = 64
_K = 8192
_C = 1000
_KR = 256
_NRARE = 4
_TEMP = 1.5
_EPS = 1e-8
_INV_LOG_C = 1.0 / math.log(float(_C))

_KBLK = 1024
_BBLK = 512


def _proj_body(p_ref, w_ref, a_ref):
    p = p_ref[...]
    pn = p * jax.lax.rsqrt(jnp.sum(p * p, axis=1, keepdims=True) + 1e-12)
    part = jax.lax.dot_general(
        pn, w_ref[...], (((0,), (0,)), ((), ())),
        preferred_element_type=jnp.float32)

    @pl.when(pl.program_id(0) == 0)
    def _init():
        a_ref[...] = part

    @pl.when(pl.program_id(0) != 0)
    def _acc():
        a_ref[...] += part


def _main_body(x_ref, a_ref, r_ref, g_ref, out_ref):
    x = x_ref[...]
    fn = x * jax.lax.rsqrt(jnp.sum(x * x, axis=1, keepdims=True) + 1e-12)
    ls = jnp.dot(fn, a_ref[...], preferred_element_type=jnp.float32)

    z = ls * (1.0 / _TEMP)
    zm = jnp.max(z, axis=1, keepdims=True)
    ez = jnp.exp(z - zm)
    probs = ez / jnp.sum(ez, axis=1, keepdims=True)
    ent = -jnp.sum(probs * jnp.log(probs + _EPS), axis=1, keepdims=True)
    u = ent * _INV_LOG_C

    r = r_ref[...]
    rn = r * jax.lax.rsqrt(jnp.sum(r * r, axis=1, keepdims=True) + 1e-12)
    s = jax.lax.dot_general(
        fn, rn, (((1,), (1,)), ((), ())),
        preferred_element_type=jnp.float32)

    col = jax.lax.broadcasted_iota(jnp.int32, ls.shape, 1)
    rad = jnp.zeros_like(ls)
    for i in range(_NRARE):
        mi = jnp.max(s[:, i * _KR:(i + 1) * _KR], axis=1, keepdims=True)
        rad = rad + jnp.where(col == i, mi * g_ref[0, i], 0.0)

    out_ref[...] = ls + u * rad


def kernel(x, shared_prototypes, W_shared_to_class, rare_prototypes, rarity_factor):
    proj = pl.pallas_call(
        _proj_body,
        grid=(_K // _KBLK,),
        in_specs=[
            pl.BlockSpec((_KBLK, _D), lambda i: (i, 0)),
            pl.BlockSpec((_KBLK, _C), lambda i: (i, 0)),
        ],
        out_specs=pl.BlockSpec((_D, _C), lambda i: (0, 0)),
        out_shape=jax.ShapeDtypeStruct((_D, _C), jnp.float32),
    )(shared_prototypes, W_shared_to_class)

    rare_flat = rare_prototypes.reshape(_NRARE * _KR, _D)
    gates = rarity_factor.reshape(1, _C)

    logits = pl.pallas_call(
        _main_body,
        grid=(_B // _BBLK,),
        in_specs=[
            pl.BlockSpec((_BBLK, _D), lambda i: (i, 0)),
            pl.BlockSpec((_D, _C), lambda i: (0, 0)),
            pl.BlockSpec((_NRARE * _KR, _D), lambda i: (0, 0)),
            pl.BlockSpec((1, _C), lambda i: (0, 0)),
        ],
        out_specs=pl.BlockSpec((_BBLK, _C), lambda i: (i, 0)),
        out_shape=jax.ShapeDtypeStruct((_B, _C), jnp.float32),
    )(x, proj, rare_flat, gates)

    return logits
